# baseline (speedup 1.0000x reference)
import functools

import jax
import jax.numpy as jnp
from jax import lax
from jax.experimental import pallas as pl
from jax.experimental.pallas import tpu as pltpu
from jax.experimental.pallas import tpu_sc as plsc

N = 8192
D = 768
E = 8
O = 768
TOPK = 2
BAL = 0.01

BN = 1024
GRID = N // BN

NW = 32
TOK_W = N // NW
NGRP = TOK_W // 16


def _logits_body(x_ref, wr_ref, br_ref, out_ref):
    out_ref[...] = (
        jnp.dot(x_ref[...], wr_ref[...], preferred_element_type=jnp.float32)
        + br_ref[...]
    )


def _route_sc_body(logits_hbm, coef_hbm, pis_hbm, lg_v, cf_v, pi_v):
    wid = lax.axis_index("s") * 2 + lax.axis_index("c")
    base = wid * TOK_W * E
    pltpu.sync_copy(logits_hbm.at[pl.ds(base, TOK_W * E)], lg_v)

    zero = jnp.zeros((16,), jnp.float32)
    for e in range(E):
        pi_v[pl.ds(e * 16, 16)] = zero

    lane8 = lax.iota(jnp.int32, 16) * E
    evecs = [jnp.full((16,), e, dtype=jnp.int32) for e in range(E)]

    for t in range(NGRP):
        flat = lane8 + (t * 16 * E)
        v = [plsc.load_gather(lg_v, [flat + evecs[e]]) for e in range(E)]
        m = v[0]
        for e in range(1, E):
            m = jnp.maximum(m, v[e])
        ex = [jnp.exp(v[e] - m) for e in range(E)]
        s = ex[0]
        for e in range(1, E):
            s = s + ex[e]
        p = [ex[e] / s for e in range(E)]
        for e in range(E):
            sl = pl.ds(e * 16, 16)
            pi_v[sl] = pi_v[sl] + p[e]
        best = p[0]
        bi = jnp.zeros((16,), jnp.int32)
        for e in range(1, E):
            upd = p[e] > best
            best = jnp.where(upd, p[e], best)
            bi = jnp.where(upd, evecs[e], bi)
        best2 = jnp.full((16,), -1.0, dtype=jnp.float32)
        bi2 = jnp.zeros((16,), jnp.int32)
        for e in range(E):
            cand = jnp.where(bi == evecs[e], -1.0, p[e])
            upd = cand > best2
            best2 = jnp.where(upd, cand, best2)
            bi2 = jnp.where(upd, evecs[e], bi2)
        denom = jnp.maximum(best + best2, 1e-9)
        w1 = best / denom
        w2 = best2 / denom
        for e in range(E):
            coef_e = (jnp.where(bi == evecs[e], w1, 0.0)
                      + jnp.where(bi2 == evecs[e], w2, 0.0))
            plsc.store_scatter(cf_v, [flat + evecs[e]], coef_e)

    pltpu.sync_copy(cf_v, coef_hbm.at[pl.ds(base, TOK_W * E)])
    pltpu.sync_copy(pi_v, pis_hbm.at[pl.ds(wid * E * 16, E * 16)])


_route_sc = functools.partial(
    pl.kernel,
    mesh=plsc.VectorSubcoreMesh(core_axis_name="c", subcore_axis_name="s"),
    compiler_params=pltpu.CompilerParams(needs_layout_passes=False),
    out_type=[
        jax.ShapeDtypeStruct((N * E,), jnp.float32),
        jax.ShapeDtypeStruct((NW * E * 16,), jnp.float32),
    ],
    scratch_types=[
        pltpu.VMEM((TOK_W * E,), jnp.float32),
        pltpu.VMEM((TOK_W * E,), jnp.float32),
        pltpu.VMEM((E * 16,), jnp.float32),
    ],
)(_route_sc_body)


def _combine_body(x_ref, coef_ref, we_ref, be_ref, pis_ref, out_ref, loss_ref):
    i = pl.program_id(0)
    x = x_ref[...]
    coef = coef_ref[...]
    acc = jnp.dot(coef, be_ref[...], preferred_element_type=jnp.float32)
    for e in range(E):
        acc = acc + coef[:, e:e + 1] * jnp.dot(
            x, we_ref[e], preferred_element_type=jnp.float32)
    out_ref[...] = acc

    @pl.when(i == GRID - 1)
    def _():
        pis = pis_ref[...]
        lane = jax.lax.broadcasted_iota(jnp.int32, (NW, E * 16), 1)
        ent = jnp.float32(0.0)
        for e in range(E):
            pi_e = jnp.sum(jnp.where(lane // 16 == e, pis, 0.0)) / N
            ent = ent + pi_e * jnp.log(jnp.maximum(pi_e, 1e-9))
        loss_ref[...] = (BAL * (ent + jnp.log(jnp.float32(E)))).reshape(1, 1)


@jax.jit
def _moe(features, Wr, br2, We, be):
    logits = pl.pallas_call(
        _logits_body,
        grid=(GRID,),
        in_specs=[
            pl.BlockSpec((BN, D), lambda i: (i, 0)),
            pl.BlockSpec((D, E), lambda i: (0, 0)),
            pl.BlockSpec((1, E), lambda i: (0, 0)),
        ],
        out_specs=pl.BlockSpec((BN, E), lambda i: (i, 0)),
        out_shape=jax.ShapeDtypeStruct((N, E), jnp.float32),
    )(features, Wr, br2)

    coef_flat, pis_flat = _route_sc(logits.reshape(N * E))
    coef = coef_flat.reshape(N, E)
    pis = pis_flat.reshape(NW, E * 16)

    out, loss = pl.pallas_call(
        _combine_body,
        grid=(GRID,),
        in_specs=[
            pl.BlockSpec((BN, D), lambda i: (i, 0)),
            pl.BlockSpec((BN, E), lambda i: (i, 0)),
            pl.BlockSpec((E, D, O), lambda i: (0, 0, 0)),
            pl.BlockSpec((E, O), lambda i: (0, 0)),
            pl.BlockSpec((NW, E * 16), lambda i: (0, 0)),
        ],
        out_specs=[
            pl.BlockSpec((BN, O), lambda i: (i, 0)),
            pl.BlockSpec((1, 1), lambda i: (0, 0)),
        ],
        out_shape=[
            jax.ShapeDtypeStruct((N, O), jnp.float32),
            jax.ShapeDtypeStruct((1, 1), jnp.float32),
        ],
    )(features, coef, We, be, pis)
    return out, loss[0, 0]


def kernel(features, Wr, br, We, be):
    return _moe(features, Wr, br.reshape(1, E), We, be)


# speedup vs baseline: 1.0007x; 1.0007x over previous
import functools

import jax
import jax.numpy as jnp
from jax import lax
from jax.experimental import pallas as pl
from jax.experimental.pallas import tpu as pltpu
from jax.experimental.pallas import tpu_sc as plsc

N = 8192
D = 768
E = 8
O = 768
TOPK = 2
BAL = 0.01

BN = 1024
GRID = N // BN

NW = 32
TOK_W = N // NW
NGRP = TOK_W // 16


def _logits_body(x_ref, wr_ref, br_ref, out_ref):
    out_ref[...] = (
        jnp.dot(x_ref[...], wr_ref[...], preferred_element_type=jnp.float32)
        + br_ref[...]
    )


def _route_sc_body(logits_hbm, coef_hbm, pis_hbm, lg_v, cf_v, pi_v):
    wid = lax.axis_index("s") * 2 + lax.axis_index("c")
    base = wid * TOK_W * E
    pltpu.sync_copy(logits_hbm.at[pl.ds(base, TOK_W * E)], lg_v)

    zero = jnp.zeros((16,), jnp.float32)
    for e in range(E):
        pi_v[pl.ds(e * 16, 16)] = zero

    lane8 = lax.iota(jnp.int32, 16) * E
    evecs = [jnp.full((16,), e, dtype=jnp.int32) for e in range(E)]

    for t in range(NGRP):
        flat = lane8 + (t * 16 * E)
        v = [plsc.load_gather(lg_v, [flat + evecs[e]]) for e in range(E)]
        m = v[0]
        for e in range(1, E):
            m = jnp.maximum(m, v[e])
        ex = [jnp.exp(v[e] - m) for e in range(E)]
        s = ex[0]
        for e in range(1, E):
            s = s + ex[e]
        p = [ex[e] / s for e in range(E)]
        for e in range(E):
            sl = pl.ds(e * 16, 16)
            pi_v[sl] = pi_v[sl] + p[e]
        best = p[0]
        bi = jnp.zeros((16,), jnp.int32)
        for e in range(1, E):
            upd = p[e] > best
            best = jnp.where(upd, p[e], best)
            bi = jnp.where(upd, evecs[e], bi)
        best2 = jnp.full((16,), -1.0, dtype=jnp.float32)
        bi2 = jnp.zeros((16,), jnp.int32)
        for e in range(E):
            cand = jnp.where(bi == evecs[e], -1.0, p[e])
            upd = cand > best2
            best2 = jnp.where(upd, cand, best2)
            bi2 = jnp.where(upd, evecs[e], bi2)
        denom = jnp.maximum(best + best2, 1e-9)
        w1 = best / denom
        w2 = best2 / denom
        for e in range(E):
            coef_e = (jnp.where(bi == evecs[e], w1, 0.0)
                      + jnp.where(bi2 == evecs[e], w2, 0.0))
            plsc.store_scatter(cf_v, [flat + evecs[e]], coef_e)

    pltpu.sync_copy(cf_v, coef_hbm.at[pl.ds(base, TOK_W * E)])
    pltpu.sync_copy(pi_v, pis_hbm.at[pl.ds(wid * E * 16, E * 16)])


_route_sc = functools.partial(
    pl.kernel,
    mesh=plsc.VectorSubcoreMesh(core_axis_name="c", subcore_axis_name="s"),
    compiler_params=pltpu.CompilerParams(needs_layout_passes=False,
                                         skip_device_barrier=True),
    out_type=[
        jax.ShapeDtypeStruct((N * E,), jnp.float32),
        jax.ShapeDtypeStruct((NW * E * 16,), jnp.float32),
    ],
    scratch_types=[
        pltpu.VMEM((TOK_W * E,), jnp.float32),
        pltpu.VMEM((TOK_W * E,), jnp.float32),
        pltpu.VMEM((E * 16,), jnp.float32),
    ],
)(_route_sc_body)


def _combine_body(x_ref, coef_ref, we_ref, be_ref, pis_ref, out_ref, loss_ref):
    i = pl.program_id(0)
    x = x_ref[...]
    coef = coef_ref[...]
    acc = jnp.dot(coef, be_ref[...], preferred_element_type=jnp.float32)
    for e in range(E):
        acc = acc + coef[:, e:e + 1] * jnp.dot(
            x, we_ref[e], preferred_element_type=jnp.float32)
    out_ref[...] = acc

    @pl.when(i == GRID - 1)
    def _():
        pis = pis_ref[...]
        lane = jax.lax.broadcasted_iota(jnp.int32, (NW, E * 16), 1)
        ent = jnp.float32(0.0)
        for e in range(E):
            pi_e = jnp.sum(jnp.where(lane // 16 == e, pis, 0.0)) / N
            ent = ent + pi_e * jnp.log(jnp.maximum(pi_e, 1e-9))
        loss_ref[...] = (BAL * (ent + jnp.log(jnp.float32(E)))).reshape(1, 1)


@jax.jit
def _moe(features, Wr, br2, We, be):
    logits = pl.pallas_call(
        _logits_body,
        grid=(GRID,),
        in_specs=[
            pl.BlockSpec((BN, D), lambda i: (i, 0)),
            pl.BlockSpec((D, E), lambda i: (0, 0)),
            pl.BlockSpec((1, E), lambda i: (0, 0)),
        ],
        out_specs=pl.BlockSpec((BN, E), lambda i: (i, 0)),
        out_shape=jax.ShapeDtypeStruct((N, E), jnp.float32),
    )(features, Wr, br2)

    coef_flat, pis_flat = _route_sc(logits.reshape(N * E))
    coef = coef_flat.reshape(N, E)
    pis = pis_flat.reshape(NW, E * 16)

    out, loss = pl.pallas_call(
        _combine_body,
        grid=(GRID,),
        in_specs=[
            pl.BlockSpec((BN, D), lambda i: (i, 0)),
            pl.BlockSpec((BN, E), lambda i: (i, 0)),
            pl.BlockSpec((E, D, O), lambda i: (0, 0, 0)),
            pl.BlockSpec((E, O), lambda i: (0, 0)),
            pl.BlockSpec((NW, E * 16), lambda i: (0, 0)),
        ],
        out_specs=[
            pl.BlockSpec((BN, O), lambda i: (i, 0)),
            pl.BlockSpec((1, 1), lambda i: (0, 0)),
        ],
        out_shape=[
            jax.ShapeDtypeStruct((N, O), jnp.float32),
            jax.ShapeDtypeStruct((1, 1), jnp.float32),
        ],
    )(features, coef, We, be, pis)
    return out, loss[0, 0]


def kernel(features, Wr, br, We, be):
    return _moe(features, Wr, br.reshape(1, E), We, be)


# speedup vs baseline: 1.0008x; 1.0001x over previous
import functools

import jax
import jax.numpy as jnp
from jax import lax
from jax.experimental import pallas as pl
from jax.experimental.pallas import tpu as pltpu
from jax.experimental.pallas import tpu_sc as plsc

N = 8192
D = 768
E = 8
O = 768
TOPK = 2
BAL = 0.01

BN = 1024
GRID = N // BN

NW = 32
TOK_W = N // NW
NGRP = TOK_W // 16


def _logits_body(x_ref, wr_ref, br_ref, out_ref):
    out_ref[...] = (
        jnp.dot(x_ref[...], wr_ref[...], preferred_element_type=jnp.float32)
        + br_ref[...]
    )


def _route_sc_body(logits_hbm, coef_hbm, pis_hbm, lg_v, cf_v, pi_v):
    wid = lax.axis_index("s") * 2 + lax.axis_index("c")
    base = wid * TOK_W * E
    pltpu.sync_copy(logits_hbm.at[pl.ds(base, TOK_W * E)], lg_v)

    zero = jnp.zeros((16,), jnp.float32)
    for e in range(E):
        pi_v[pl.ds(e * 16, 16)] = zero

    lane8 = lax.iota(jnp.int32, 16) * E
    evecs = [jnp.full((16,), e, dtype=jnp.int32) for e in range(E)]

    for t in range(NGRP):
        flat = lane8 + (t * 16 * E)
        v = [plsc.load_gather(lg_v, [flat + evecs[e]]) for e in range(E)]
        m = v[0]
        for e in range(1, E):
            m = jnp.maximum(m, v[e])
        ex = [jnp.exp(v[e] - m) for e in range(E)]
        s = ex[0]
        for e in range(1, E):
            s = s + ex[e]
        p = [ex[e] / s for e in range(E)]
        for e in range(E):
            sl = pl.ds(e * 16, 16)
            pi_v[sl] = pi_v[sl] + p[e]
        best = p[0]
        bi = jnp.zeros((16,), jnp.int32)
        for e in range(1, E):
            upd = p[e] > best
            best = jnp.where(upd, p[e], best)
            bi = jnp.where(upd, evecs[e], bi)
        best2 = jnp.full((16,), -1.0, dtype=jnp.float32)
        bi2 = jnp.zeros((16,), jnp.int32)
        for e in range(E):
            cand = jnp.where(bi == evecs[e], -1.0, p[e])
            upd = cand > best2
            best2 = jnp.where(upd, cand, best2)
            bi2 = jnp.where(upd, evecs[e], bi2)
        denom = jnp.maximum(best + best2, 1e-9)
        w1 = best / denom
        w2 = best2 / denom
        for e in range(E):
            coef_e = (jnp.where(bi == evecs[e], w1, 0.0)
                      + jnp.where(bi2 == evecs[e], w2, 0.0))
            plsc.store_scatter(cf_v, [flat + evecs[e]], coef_e)

    pltpu.sync_copy(cf_v, coef_hbm.at[pl.ds(base, TOK_W * E)])
    pltpu.sync_copy(pi_v, pis_hbm.at[pl.ds(wid * E * 16, E * 16)])


_route_sc = functools.partial(
    pl.kernel,
    mesh=plsc.VectorSubcoreMesh(core_axis_name="c", subcore_axis_name="s"),
    compiler_params=pltpu.CompilerParams(needs_layout_passes=False),
    out_type=[
        jax.ShapeDtypeStruct((N * E,), jnp.float32),
        jax.ShapeDtypeStruct((NW * E * 16,), jnp.float32),
    ],
    scratch_types=[
        pltpu.VMEM((TOK_W * E,), jnp.float32),
        pltpu.VMEM((TOK_W * E,), jnp.float32),
        pltpu.VMEM((E * 16,), jnp.float32),
    ],
)(_route_sc_body)


def _combine_body(x_ref, coef_ref, we_ref, be_ref, pis_ref, out_ref, loss_ref):
    i = pl.program_id(0)
    x = x_ref[...]
    coef = coef_ref[...]
    acc = jnp.dot(coef, be_ref[...], preferred_element_type=jnp.float32)
    for e in range(E):
        acc = acc + coef[:, e:e + 1] * jnp.dot(
            x, we_ref[e], preferred_element_type=jnp.float32)
    out_ref[...] = acc

    @pl.when(i == GRID - 1)
    def _():
        pis = pis_ref[...]
        lane = jax.lax.broadcasted_iota(jnp.int32, (NW, E * 16), 1)
        ent = jnp.float32(0.0)
        for e in range(E):
            pi_e = jnp.sum(jnp.where(lane // 16 == e, pis, 0.0)) / N
            ent = ent + pi_e * jnp.log(jnp.maximum(pi_e, 1e-9))
        loss_ref[...] = (BAL * (ent + jnp.log(jnp.float32(E)))).reshape(1, 1)


@jax.jit
def _moe(features, Wr, br2, We, be):
    logits = pl.pallas_call(
        _logits_body,
        grid=(GRID,),
        in_specs=[
            pl.BlockSpec((BN, D), lambda i: (i, 0)),
            pl.BlockSpec((D, E), lambda i: (0, 0)),
            pl.BlockSpec((1, E), lambda i: (0, 0)),
        ],
        out_specs=pl.BlockSpec((BN, E), lambda i: (i, 0)),
        out_shape=jax.ShapeDtypeStruct((N, E), jnp.float32),
    )(features, Wr, br2)

    coef_flat, pis_flat = _route_sc(logits.reshape(N * E))
    coef = coef_flat.reshape(N, E)
    pis = pis_flat.reshape(NW, E * 16)

    out, loss = pl.pallas_call(
        _combine_body,
        grid=(GRID,),
        in_specs=[
            pl.BlockSpec((BN, D), lambda i: (i, 0)),
            pl.BlockSpec((BN, E), lambda i: (i, 0)),
            pl.BlockSpec((E, D, O), lambda i: (0, 0, 0)),
            pl.BlockSpec((E, O), lambda i: (0, 0)),
            pl.BlockSpec((NW, E * 16), lambda i: (0, 0)),
        ],
        out_specs=[
            pl.BlockSpec((BN, O), lambda i: (i, 0)),
            pl.BlockSpec((1, 1), lambda i: (0, 0)),
        ],
        out_shape=[
            jax.ShapeDtypeStruct((N, O), jnp.float32),
            jax.ShapeDtypeStruct((1, 1), jnp.float32),
        ],
    )(features, coef, We, be, pis)
    return out, loss[0, 0]


def kernel(features, Wr, br, We, be):
    return _moe(features, Wr, br.reshape(1, E), We, be)
